# baseline (speedup 1.0000x reference)
import functools

import jax
import jax.numpy as jnp
from jax import lax
from jax.experimental import pallas as pl
from jax.experimental.pallas import tpu as pltpu
from jax.experimental.pallas import tpu_sc as plsc


def _dist_argmin_kernel(x_ref, cb_ref, idx_ref, h_ref):
    @pl.when(pl.program_id(0) == 0)
    def _():
        c = cb_ref[...]
        h_ref[...] = 0.5 * jnp.sum(c * c, axis=1, keepdims=True)

    c, h, w = x_ref.shape[1:]
    xb = x_ref[0].reshape(c, h * w)
    dots = lax.dot_general(
        cb_ref[...], xb, (((1,), (0,)), ((), ())),
        preferred_element_type=jnp.float32,
    )
    first = jnp.argmin(h_ref[...] - dots, axis=0)
    idx_ref[0, 0, :] = first.astype(jnp.int32)


def _dist_argmin(x, cb):
    B, C, H, W = x.shape
    N = cb.shape[0]
    return pl.pallas_call(
        _dist_argmin_kernel,
        grid=(B,),
        in_specs=[
            pl.BlockSpec((1, C, H, W), lambda i: (i, 0, 0, 0)),
            pl.BlockSpec((N, C), lambda i: (0, 0)),
        ],
        out_specs=pl.BlockSpec((1, 1, H * W), lambda i: (i, 0, 0)),
        out_shape=jax.ShapeDtypeStruct((B, 1, H * W), jnp.int32),
        scratch_shapes=[pltpu.VMEM((N, 1), jnp.float32)],
    )(x, cb)


def _sc_gather(table, idx):
    V, D = table.shape
    B = idx.shape[0]
    info = plsc.get_sparse_core_info()
    nw = info.num_cores * info.num_subcores
    b_per_w = B // nw
    mesh = plsc.VectorSubcoreMesh(core_axis_name="c", subcore_axis_name="s")

    @functools.partial(
        pl.kernel,
        mesh=mesh,
        out_type=jax.ShapeDtypeStruct((B, D), jnp.float32),
        scratch_types=[
            pltpu.VMEM((b_per_w,), jnp.int32),
            pltpu.VMEM((b_per_w, D), jnp.float32),
            pltpu.SemaphoreType.DMA,
        ],
    )
    def gather(table_hbm, idx_hbm, out_hbm, idx_v, rows_v, sem):
        wid = lax.axis_index("s") * info.num_cores + lax.axis_index("c")
        base = wid * b_per_w
        pltpu.sync_copy(idx_hbm.at[pl.ds(base, b_per_w)], idx_v)
        pltpu.async_copy(table_hbm.at[idx_v], rows_v, sem).wait()
        pltpu.sync_copy(rows_v, out_hbm.at[pl.ds(base, b_per_w)])

    return gather(table, idx)


def kernel(x, codebook):
    B, C, H, W = x.shape
    idx = _dist_argmin(x, codebook).reshape(B * H * W)
    rows = _sc_gather(codebook, idx)
    out = rows.reshape(B, H * W, C)
    return jnp.transpose(out, (0, 2, 1)).reshape(B, C, H, W)


# speedup vs baseline: 1.0182x; 1.0182x over previous
import functools

import jax
import jax.numpy as jnp
from jax import lax
from jax.experimental import pallas as pl
from jax.experimental.pallas import tpu as pltpu
from jax.experimental.pallas import tpu_sc as plsc


def _dist_argmin_kernel(x_ref, cb_ref, idx_ref, h_ref):
    @pl.when(pl.program_id(0) == 0)
    def _():
        c = cb_ref[...]
        h_ref[...] = 0.5 * jnp.sum(c * c, axis=1, keepdims=True)

    xb = x_ref[0]
    dots = lax.dot_general(
        cb_ref[...], xb, (((1,), (0,)), ((), ())),
        preferred_element_type=jnp.float32,
    )
    first = jnp.argmin(h_ref[...] - dots, axis=0)
    idx_ref[0, 0, :] = first.astype(jnp.int32)


def _dist_argmin(x3, cb):
    B, C, M = x3.shape
    N = cb.shape[0]
    return pl.pallas_call(
        _dist_argmin_kernel,
        grid=(B,),
        in_specs=[
            pl.BlockSpec((1, C, M), lambda i: (i, 0, 0)),
            pl.BlockSpec((N, C), lambda i: (0, 0)),
        ],
        out_specs=pl.BlockSpec((1, 1, M), lambda i: (i, 0, 0)),
        out_shape=jax.ShapeDtypeStruct((B, 1, M), jnp.int32),
        scratch_shapes=[pltpu.VMEM((N, 1), jnp.float32)],
    )(x3, cb)


def _sc_gather(table, idx):
    V, D = table.shape
    B = idx.shape[0]
    info = plsc.get_sparse_core_info()
    nw = info.num_cores * info.num_subcores
    b_per_w = B // nw
    mesh = plsc.VectorSubcoreMesh(core_axis_name="c", subcore_axis_name="s")

    @functools.partial(
        pl.kernel,
        mesh=mesh,
        out_type=jax.ShapeDtypeStruct((B, D), jnp.float32),
        scratch_types=[
            pltpu.VMEM((b_per_w,), jnp.int32),
            pltpu.VMEM((b_per_w, D), jnp.float32),
            pltpu.SemaphoreType.DMA,
        ],
    )
    def gather(table_hbm, idx_hbm, out_hbm, idx_v, rows_v, sem):
        wid = lax.axis_index("s") * info.num_cores + lax.axis_index("c")
        base = wid * b_per_w
        pltpu.sync_copy(idx_hbm.at[pl.ds(base, b_per_w)], idx_v)
        pltpu.async_copy(table_hbm.at[idx_v], rows_v, sem).wait()
        pltpu.sync_copy(rows_v, out_hbm.at[pl.ds(base, b_per_w)])

    return gather(table, idx)


def kernel(x, codebook):
    B, C, H, W = x.shape
    x3 = x.reshape(B, C, H * W)
    splits = 2
    bs = B // splits
    outs = []
    for i in range(splits):
        idx = _dist_argmin(x3[i * bs : (i + 1) * bs], codebook)
        rows = _sc_gather(codebook, idx.reshape(bs * H * W))
        out = rows.reshape(bs, H * W, C)
        outs.append(jnp.transpose(out, (0, 2, 1)).reshape(bs, C, H, W))
    return jnp.concatenate(outs, axis=0)


# speedup vs baseline: 1.1715x; 1.1505x over previous
import functools

import jax
import jax.numpy as jnp
from jax import lax
from jax.experimental import pallas as pl
from jax.experimental.pallas import tpu as pltpu
from jax.experimental.pallas import tpu_sc as plsc


def _dist_argmin_kernel(x_ref, cb_ref, idx_ref, h_ref):
    @pl.when(pl.program_id(0) == 0)
    def _():
        c = cb_ref[...]
        h_ref[...] = 0.5 * jnp.sum(c * c, axis=1, keepdims=True)

    xb = x_ref[0]
    dots = lax.dot_general(
        cb_ref[...], xb, (((1,), (0,)), ((), ())),
        preferred_element_type=jnp.float32,
    )
    first = jnp.argmin(h_ref[...] - dots, axis=0)
    idx_ref[0, 0, :] = first.astype(jnp.int32)


def _dist_argmin(x3, cb):
    B, C, M = x3.shape
    N = cb.shape[0]
    return pl.pallas_call(
        _dist_argmin_kernel,
        grid=(B,),
        in_specs=[
            pl.BlockSpec((1, C, M), lambda i: (i, 0, 0)),
            pl.BlockSpec((N, C), lambda i: (0, 0)),
        ],
        out_specs=pl.BlockSpec((1, 1, M), lambda i: (i, 0, 0)),
        out_shape=jax.ShapeDtypeStruct((B, 1, M), jnp.int32),
        scratch_shapes=[pltpu.VMEM((N, 1), jnp.float32)],
    )(x3, cb)


def _sc_gather(table, idx, nc=3):
    V, D = table.shape
    B = idx.shape[0]
    info = plsc.get_sparse_core_info()
    nw = info.num_cores * info.num_subcores
    b_per_w = B // nw
    ck = b_per_w // nc
    mesh = plsc.VectorSubcoreMesh(core_axis_name="c", subcore_axis_name="s")

    @functools.partial(
        pl.kernel,
        mesh=mesh,
        out_type=jax.ShapeDtypeStruct((B, D), jnp.float32),
        scratch_types=[
            pltpu.VMEM((b_per_w,), jnp.int32),
            pltpu.VMEM((b_per_w, D), jnp.float32),
        ]
        + [pltpu.SemaphoreType.DMA] * (2 * nc),
    )
    def gather(table_hbm, idx_hbm, out_hbm, idx_v, rows_v, *sems):
        gsems, wsems = sems[:nc], sems[nc:]
        wid = lax.axis_index("s") * info.num_cores + lax.axis_index("c")
        base = wid * b_per_w
        pltpu.sync_copy(idx_hbm.at[pl.ds(base, b_per_w)], idx_v)
        gets = [
            pltpu.async_copy(
                table_hbm.at[idx_v.at[pl.ds(c * ck, ck)]],
                rows_v.at[pl.ds(c * ck, ck)],
                gsems[c],
            )
            for c in range(nc)
        ]
        puts = []
        for c in range(nc):
            gets[c].wait()
            puts.append(
                pltpu.async_copy(
                    rows_v.at[pl.ds(c * ck, ck)],
                    out_hbm.at[pl.ds(base + c * ck, ck)],
                    wsems[c],
                )
            )
        for p in puts:
            p.wait()

    return gather(table, idx)


def kernel(x, codebook):
    B, C, H, W = x.shape
    x3 = x.reshape(B, C, H * W)
    idx = _dist_argmin(x3, codebook).reshape(B * H * W)
    rows = _sc_gather(codebook, idx)
    out = rows.reshape(B, H * W, C)
    return jnp.transpose(out, (0, 2, 1)).reshape(B, C, H, W)


# speedup vs baseline: 1.1740x; 1.0022x over previous
import functools

import jax
import jax.numpy as jnp
from jax import lax
from jax.experimental import pallas as pl
from jax.experimental.pallas import tpu as pltpu
from jax.experimental.pallas import tpu_sc as plsc


def _dist_argmin_kernel(x_ref, cb_ref, idx_ref, cbt_ref, h_ref):
    @pl.when(pl.program_id(0) == 0)
    def _():
        ct = jnp.transpose(cb_ref[...], (1, 0))
        cbt_ref[...] = ct
        h_ref[...] = 0.5 * jnp.sum(ct * ct, axis=0, keepdims=True)

    xb = x_ref[0]
    dots = lax.dot_general(
        xb, cbt_ref[...], (((0,), (0,)), ((), ())),
        preferred_element_type=jnp.float32,
    )
    first = jnp.argmin(h_ref[...] - dots, axis=1)
    idx_ref[0, 0, :] = first.astype(jnp.int32)


def _dist_argmin(x3, cb):
    B, C, M = x3.shape
    N = cb.shape[0]
    return pl.pallas_call(
        _dist_argmin_kernel,
        grid=(B,),
        in_specs=[
            pl.BlockSpec((1, C, M), lambda i: (i, 0, 0)),
            pl.BlockSpec((N, C), lambda i: (0, 0)),
        ],
        out_specs=pl.BlockSpec((1, 1, M), lambda i: (i, 0, 0)),
        out_shape=jax.ShapeDtypeStruct((B, 1, M), jnp.int32),
        scratch_shapes=[
            pltpu.VMEM((C, N), jnp.float32),
            pltpu.VMEM((1, N), jnp.float32),
        ],
    )(x3, cb)


def _sc_gather(table, idx, nc=3):
    V, D = table.shape
    B = idx.shape[0]
    info = plsc.get_sparse_core_info()
    nw = info.num_cores * info.num_subcores
    b_per_w = B // nw
    ck = b_per_w // nc
    mesh = plsc.VectorSubcoreMesh(core_axis_name="c", subcore_axis_name="s")

    @functools.partial(
        pl.kernel,
        mesh=mesh,
        out_type=jax.ShapeDtypeStruct((B, D), jnp.float32),
        scratch_types=[
            pltpu.VMEM((b_per_w,), jnp.int32),
            pltpu.VMEM((b_per_w, D), jnp.float32),
        ]
        + [pltpu.SemaphoreType.DMA] * (2 * nc),
    )
    def gather(table_hbm, idx_hbm, out_hbm, idx_v, rows_v, *sems):
        gsems, wsems = sems[:nc], sems[nc:]
        wid = lax.axis_index("s") * info.num_cores + lax.axis_index("c")
        base = wid * b_per_w
        pltpu.sync_copy(idx_hbm.at[pl.ds(base, b_per_w)], idx_v)
        gets = [
            pltpu.async_copy(
                table_hbm.at[idx_v.at[pl.ds(c * ck, ck)]],
                rows_v.at[pl.ds(c * ck, ck)],
                gsems[c],
            )
            for c in range(nc)
        ]
        puts = []
        for c in range(nc):
            gets[c].wait()
            puts.append(
                pltpu.async_copy(
                    rows_v.at[pl.ds(c * ck, ck)],
                    out_hbm.at[pl.ds(base + c * ck, ck)],
                    wsems[c],
                )
            )
        for p in puts:
            p.wait()

    return gather(table, idx)


def kernel(x, codebook):
    B, C, H, W = x.shape
    x3 = x.reshape(B, C, H * W)
    idx = _dist_argmin(x3, codebook).reshape(B * H * W)
    rows = _sc_gather(codebook, idx)
    out = rows.reshape(B, H * W, C)
    return jnp.transpose(out, (0, 2, 1)).reshape(B, C, H, W)


# speedup vs baseline: 1.1797x; 1.0049x over previous
import functools

import jax
import jax.numpy as jnp
from jax import lax
from jax.experimental import pallas as pl
from jax.experimental.pallas import tpu as pltpu
from jax.experimental.pallas import tpu_sc as plsc


def _dist_argmin_kernel(x_ref, cb_ref, idx_ref, h_ref):
    @pl.when(pl.program_id(0) == 0)
    def _():
        c = cb_ref[...]
        h_ref[...] = 0.5 * jnp.sum(c * c, axis=1, keepdims=True)

    xb = x_ref[0]
    dots = lax.dot_general(
        cb_ref[...], xb, (((1,), (0,)), ((), ())),
        preferred_element_type=jnp.float32,
    )
    first = jnp.argmin(h_ref[...] - dots, axis=0)
    idx_ref[0, 0, :] = first.astype(jnp.int32)


def _dist_argmin(x3, cb):
    B, C, M = x3.shape
    N = cb.shape[0]
    return pl.pallas_call(
        _dist_argmin_kernel,
        grid=(B,),
        in_specs=[
            pl.BlockSpec((1, C, M), lambda i: (i, 0, 0)),
            pl.BlockSpec((N, C), lambda i: (0, 0)),
        ],
        out_specs=pl.BlockSpec((1, 1, M), lambda i: (i, 0, 0)),
        out_shape=jax.ShapeDtypeStruct((B, 1, M), jnp.int32),
        scratch_shapes=[pltpu.VMEM((N, 1), jnp.float32)],
    )(x3, cb)


def _sc_gather(table, idx):
    V, D = table.shape
    B = idx.shape[0]
    info = plsc.get_sparse_core_info()
    nw = info.num_cores * info.num_subcores
    b_per_w = B // nw
    mesh = plsc.VectorSubcoreMesh(core_axis_name="c", subcore_axis_name="s")

    @functools.partial(
        pl.kernel,
        mesh=mesh,
        out_type=jax.ShapeDtypeStruct((B, D), jnp.float32),
        scratch_types=[
            pltpu.VMEM((b_per_w,), jnp.int32),
            pltpu.VMEM((b_per_w, D), jnp.float32),
            pltpu.SemaphoreType.DMA,
        ],
    )
    def gather(table_hbm, idx_hbm, out_hbm, idx_v, rows_v, sem):
        wid = lax.axis_index("s") * info.num_cores + lax.axis_index("c")
        base = wid * b_per_w
        pltpu.sync_copy(idx_hbm.at[pl.ds(base, b_per_w)], idx_v)
        pltpu.async_copy(table_hbm.at[idx_v], rows_v, sem).wait()
        pltpu.sync_copy(rows_v, out_hbm.at[pl.ds(base, b_per_w)])

    return gather(table, idx)


def kernel(x, codebook):
    B, C, H, W = x.shape
    x3 = x.reshape(B, C, H * W)
    idx = _dist_argmin(x3, codebook).reshape(B * H * W)
    rows = _sc_gather(codebook, idx)
    out = rows.reshape(B, H * W, C)
    return jnp.transpose(out, (0, 2, 1)).reshape(B, C, H, W)


# speedup vs baseline: 1.2329x; 1.0451x over previous
import functools

import jax
import jax.numpy as jnp
from jax import lax
from jax.experimental import pallas as pl
from jax.experimental.pallas import tpu as pltpu
from jax.experimental.pallas import tpu_sc as plsc


def _dist_argmin_kernel(x_ref, cb_ref, idx_ref, h_ref):
    @pl.when(pl.program_id(0) == 0)
    def _():
        c = cb_ref[...]
        h_ref[...] = 0.5 * jnp.sum(c * c, axis=1, keepdims=True)

    for b in range(x_ref.shape[0]):
        xb = x_ref[b]
        dots = lax.dot_general(
            cb_ref[...], xb, (((1,), (0,)), ((), ())),
            preferred_element_type=jnp.float32,
        )
        first = jnp.argmin(h_ref[...] - dots, axis=0)
        idx_ref[b, 0, :] = first.astype(jnp.int32)


def _dist_argmin(x3, cb, bpp=2):
    B, C, M = x3.shape
    N = cb.shape[0]
    return pl.pallas_call(
        _dist_argmin_kernel,
        grid=(B // bpp,),
        in_specs=[
            pl.BlockSpec((bpp, C, M), lambda i: (i, 0, 0)),
            pl.BlockSpec((N, C), lambda i: (0, 0)),
        ],
        out_specs=pl.BlockSpec((bpp, 1, M), lambda i: (i, 0, 0)),
        out_shape=jax.ShapeDtypeStruct((B, 1, M), jnp.int32),
        scratch_shapes=[pltpu.VMEM((N, 1), jnp.float32)],
    )(x3, cb)


def _sc_gather(table, idx):
    V, D = table.shape
    B = idx.shape[0]
    info = plsc.get_sparse_core_info()
    nw = info.num_cores * info.num_subcores
    b_per_w = B // nw
    mesh = plsc.VectorSubcoreMesh(core_axis_name="c", subcore_axis_name="s")

    @functools.partial(
        pl.kernel,
        mesh=mesh,
        out_type=jax.ShapeDtypeStruct((B, D), jnp.float32),
        scratch_types=[
            pltpu.VMEM((b_per_w,), jnp.int32),
            pltpu.VMEM((b_per_w, D), jnp.float32),
            pltpu.SemaphoreType.DMA,
        ],
    )
    def gather(table_hbm, idx_hbm, out_hbm, idx_v, rows_v, sem):
        wid = lax.axis_index("s") * info.num_cores + lax.axis_index("c")
        base = wid * b_per_w
        pltpu.sync_copy(idx_hbm.at[pl.ds(base, b_per_w)], idx_v)
        pltpu.async_copy(table_hbm.at[idx_v], rows_v, sem).wait()
        pltpu.sync_copy(rows_v, out_hbm.at[pl.ds(base, b_per_w)])

    return gather(table, idx)


def kernel(x, codebook):
    B, C, H, W = x.shape
    x3 = x.reshape(B, C, H * W)
    idx = _dist_argmin(x3, codebook).reshape(B * H * W)
    rows = _sc_gather(codebook, idx)
    out = rows.reshape(B, H * W, C)
    return jnp.transpose(out, (0, 2, 1)).reshape(B, C, H, W)
